# baseline (speedup 1.0000x reference)
import functools
import math

import jax
import jax.numpy as jnp
from jax.experimental import pallas as pl
from jax.experimental.pallas import tpu as pltpu
from jax.experimental.pallas import tpu_sc as plsc

_D = 128
_SCALE = math.sqrt(float(_D))
_BW = 8


def _sc_gather(table, idx, batch, seq):
    mesh = plsc.VectorSubcoreMesh(core_axis_name="c", subcore_axis_name="s")

    @functools.partial(
        pl.kernel,
        out_type=jax.ShapeDtypeStruct((batch * seq, _D), jnp.float32),
        mesh=mesh,
        scratch_types=[pltpu.SemaphoreType.DMA],
    )
    def k(t_hbm, i_hbm, o_hbm, sem):
        def body(i_vmem, o_vmem):
            copies = [
                pltpu.async_copy(
                    t_hbm.at[i_vmem.at[j]],
                    o_vmem.at[pl.ds(seq * j, seq)],
                    sem,
                )
                for j in range(_BW)
            ]
            for c in copies:
                c.wait()

        pltpu.emit_pipeline(
            body,
            grid=(batch // _BW,),
            in_specs=[pl.BlockSpec((_BW, seq), index_map=lambda i: (i, 0))],
            out_specs=[pl.BlockSpec((_BW * seq, _D), index_map=lambda i: (i, 0))],
            core_axis_name=("c", "s"),
            dimension_semantics=(pltpu.PARALLEL,),
        )(i_hbm, o_hbm)

    return k(table, idx)


def _scale_body(f_ref, o_ref):
    o_ref[...] = f_ref[...].reshape(o_ref.shape) * _SCALE


def _tc_scale(flat, batch, seq):
    return pl.pallas_call(
        _scale_body,
        out_shape=jax.ShapeDtypeStruct((batch, seq, _D), jnp.float32),
        grid=(batch // _BW,),
        in_specs=[pl.BlockSpec((_BW * seq, _D), lambda i: (i, 0))],
        out_specs=pl.BlockSpec((_BW, seq, _D), lambda i: (i, 0, 0)),
    )(flat)


def kernel(inputs, table):
    batch, seq = inputs.shape
    flat = _sc_gather(table, inputs, batch, seq)
    return _tc_scale(flat, batch, seq)


# speedup vs baseline: 1.3334x; 1.3334x over previous
import functools
import math

import jax
import jax.numpy as jnp
from jax.experimental import pallas as pl
from jax.experimental.pallas import tpu as pltpu
from jax.experimental.pallas import tpu_sc as plsc

_D = 128
_SCALE = math.sqrt(float(_D))
_W = 128
_G = 2
_BLK = _W * _G


def _sc_gather(table, idx_flat, n):
    mesh = plsc.VectorSubcoreMesh(core_axis_name="c", subcore_axis_name="s")

    @functools.partial(
        pl.kernel,
        out_type=jax.ShapeDtypeStruct((n, _D), jnp.float32),
        mesh=mesh,
        scratch_types=[pltpu.SemaphoreType.DMA],
    )
    def k(t_hbm, i_hbm, o_hbm, sem):
        def body(i_vmem, o_vmem):
            copies = [
                pltpu.async_copy(
                    t_hbm.at[i_vmem.at[0, pl.ds(_W * j, _W)]],
                    o_vmem.at[pl.ds(_W * j, _W)],
                    sem,
                )
                for j in range(_G)
            ]
            for c in copies:
                c.wait()

        pltpu.emit_pipeline(
            body,
            grid=(n // _BLK,),
            in_specs=[pl.BlockSpec((1, _BLK), index_map=lambda i: (0, i))],
            out_specs=[pl.BlockSpec((_BLK, _D), index_map=lambda i: (i, 0))],
            core_axis_name=("c", "s"),
            dimension_semantics=(pltpu.PARALLEL,),
        )(i_hbm, o_hbm)

    return k(table, idx_flat)


def kernel(inputs, table):
    batch, seq = inputs.shape
    n = batch * seq
    flat = _sc_gather(table, inputs.reshape(1, n), n)
    return (flat * jnp.float32(_SCALE)).reshape(batch, seq, _D)


# speedup vs baseline: 1.8425x; 1.3818x over previous
import functools
import math

import jax
import jax.numpy as jnp
from jax.experimental import pallas as pl
from jax.experimental.pallas import tpu as pltpu
from jax.experimental.pallas import tpu_sc as plsc

_D = 128
_SCALE = math.sqrt(float(_D))
_BW = 8


def _sc_gather(table, idx, batch, seq):
    mesh = plsc.VectorSubcoreMesh(core_axis_name="c", subcore_axis_name="s")

    @functools.partial(
        pl.kernel,
        out_type=jax.ShapeDtypeStruct((batch, seq, _D), jnp.float32),
        mesh=mesh,
        scratch_types=[pltpu.SemaphoreType.DMA],
    )
    def k(t_hbm, i_hbm, o_hbm, sem):
        def body(i_vmem, o_vmem):
            copies = [
                pltpu.async_copy(t_hbm.at[i_vmem.at[j]], o_vmem.at[j], sem)
                for j in range(_BW)
            ]
            for c in copies:
                c.wait()

        pltpu.emit_pipeline(
            body,
            grid=(batch // _BW,),
            in_specs=[pl.BlockSpec((_BW, seq), index_map=lambda i: (i, 0))],
            out_specs=[pl.BlockSpec((_BW, seq, _D), index_map=lambda i: (i, 0, 0))],
            core_axis_name=("c", "s"),
            dimension_semantics=(pltpu.PARALLEL,),
        )(i_hbm, o_hbm)

    return k(table, idx)


def kernel(inputs, table):
    batch, seq = inputs.shape
    out = _sc_gather(table, inputs, batch, seq)
    return out * jnp.float32(_SCALE)
